# baseline (speedup 1.0000x reference)
import functools

import jax
import jax.numpy as jnp
from jax import lax
from jax.experimental import pallas as pl
from jax.experimental.pallas import tpu as pltpu
from jax.experimental.pallas import tpu_sc as plsc

N = 10000
NP = 10240
E = 320000
F = 128
FH = F // 2
NG = 64

NC = 2
NS = 16
CHUNK = 80
EPT_D = E // (NC * NS)
NCH_D = EPT_D // CHUNK
CHUNK_P = 80
NCH_P = 256
EPT_P = NCH_P * CHUNK_P
E_PAD = NS * EPT_P
PADNODE = N
RPT = NP // NS
ZROWS = 128
DEGW = 16

RB = 1024
GRID = NP // RB

_F32 = jnp.float32


def _sc_mesh():
    return plsc.VectorSubcoreMesh(core_axis_name="c", subcore_axis_name="s",
                                  num_cores=NC, num_subcores=NS)


_DEG_KW = dict(
    mesh=_sc_mesh(),
    out_type=jax.ShapeDtypeStruct((NC, NP, DEGW), _F32),
    scratch_types=[
        pltpu.VMEM((NCH_D, CHUNK), jnp.int32),
        pltpu.VMEM((CHUNK, DEGW), _F32),
        pltpu.VMEM((RPT, DEGW), _F32),
        pltpu.VMEM_SHARED((NP, DEGW), _F32),
    ],
    compiler_params=pltpu.CompilerParams(use_tc_tiling_on_sc=False),
)


def _deg_body(dst_hbm, out_hbm, dstv, onesv, zb, deg_sh):
    c = lax.axis_index("c")
    s = lax.axis_index("s")
    pltpu.sync_copy(dst_hbm.at[c, s], dstv)

    ones16 = jnp.ones((16,), _F32)
    z16 = jnp.zeros((16,), _F32)

    def fill_ones(r, carry):
        onesv[r, :] = ones16
        return carry

    lax.fori_loop(0, CHUNK, fill_ones, 0)

    def fill_zero(r, carry):
        zb[r, :] = z16
        return carry

    lax.fori_loop(0, RPT, fill_zero, 0)
    pltpu.sync_copy(zb, deg_sh.at[pl.ds(s * RPT, RPT)])
    plsc.subcore_barrier()

    def body(i, carry):
        pltpu.sync_copy(onesv, deg_sh.at[dstv.at[i]], add=True)
        return carry

    lax.fori_loop(0, NCH_D, body, 0)
    plsc.subcore_barrier()
    pltpu.sync_copy(deg_sh.at[pl.ds(s * RPT, RPT)],
                    out_hbm.at[c, pl.ds(s * RPT, RPT)])


_PROP_KW = dict(
    mesh=_sc_mesh(),
    out_type=(jax.ShapeDtypeStruct((NP, FH), _F32),
              jax.ShapeDtypeStruct((NP, FH), _F32)),
    scratch_types=[
        pltpu.VMEM((NCH_P, CHUNK_P), jnp.int32),
        pltpu.VMEM((NCH_P, CHUNK_P), jnp.int32),
    ] + [pltpu.VMEM((CHUNK_P, FH), _F32)] * 8 + [
        pltpu.VMEM((ZROWS, FH), _F32),
        pltpu.VMEM_SHARED((NP, FH), _F32),
    ] + [pltpu.SemaphoreType.DMA] * 16,
    compiler_params=pltpu.CompilerParams(use_tc_tiling_on_sc=False),
)


def _prop_body(g0_hbm, g1_hbm, src_hbm, dst_hbm, p0_hbm, p1_hbm,
             srcv, dstv, gb0, gb1, gb2, gb3, gb4, gb5, gb6, gb7, zb, p_sh,
             sg0, sg1, sg2, sg3, sg4, sg5, sg6, sg7,
             ss0, ss1, ss2, ss3, ss4, ss5, ss6, ss7):
    c = lax.axis_index("c")
    s = lax.axis_index("s")
    pltpu.sync_copy(src_hbm.at[s], srcv)
    pltpu.sync_copy(dst_hbm.at[s], dstv)

    z16 = jnp.zeros((16,), _F32)

    def fill_zero(r, carry):
        for j in range(FH // 16):
            zb[r, pl.ds(j * 16, 16)] = z16
        return carry

    lax.fori_loop(0, ZROWS, fill_zero, 0)
    for k in range(RPT // ZROWS):
        pltpu.sync_copy(zb, p_sh.at[pl.ds(s * RPT + k * ZROWS, ZROWS)])
    plsc.subcore_barrier()

    def run_half(g_hbm):
        bufs = (gb0, gb1, gb2, gb3, gb4, gb5, gb6, gb7)
        sgs = (sg0, sg1, sg2, sg3, sg4, sg5, sg6, sg7)
        sss = (ss0, ss1, ss2, ss3, ss4, ss5, ss6, ss7)

        def wait_g(buf, sem):
            pltpu.make_async_copy(g_hbm.at[srcv.at[0]], buf, sem).wait()

        def wait_s(buf, sem):
            pltpu.make_async_copy(buf, p_sh.at[dstv.at[0]], sem).wait()

        pltpu.async_copy(g_hbm.at[srcv.at[0]], bufs[0], sgs[0])
        ngrp = NCH_P // 8

        def body4(j, carry):
            i = 8 * j
            for b in range(8):
                nb = (b + 1) % 8
                if b < 7:
                    @pl.when(j > 0)
                    def _(nb=nb):
                        wait_s(bufs[nb], sss[nb])
                    pltpu.async_copy(g_hbm.at[srcv.at[i + b + 1]],
                                     bufs[nb], sgs[nb])
                else:
                    wait_s(bufs[0], sss[0])

                    @pl.when(j < ngrp - 1)
                    def _():
                        pltpu.async_copy(g_hbm.at[srcv.at[i + 8]],
                                         bufs[0], sgs[0])
                wait_g(bufs[b], sgs[b])
                pltpu.async_copy(bufs[b], p_sh.at[dstv.at[i + b]], sss[b],
                                 add=True)
            return carry

        lax.fori_loop(0, ngrp, body4, 0)
        for b in range(1, 8):
            wait_s(bufs[b], sss[b])

    @pl.when(c == 0)
    def _():
        run_half(g0_hbm)

    @pl.when(c == 1)
    def _():
        run_half(g1_hbm)

    plsc.subcore_barrier()

    @pl.when(c == 0)
    def _():
        pltpu.sync_copy(p_sh.at[pl.ds(s * RPT, RPT)],
                        p0_hbm.at[pl.ds(s * RPT, RPT)])

    @pl.when(c == 1)
    def _():
        pltpu.sync_copy(p_sh.at[pl.ds(s * RPT, RPT)],
                        p1_hbm.at[pl.ds(s * RPT, RPT)])


_deg_sc = pl.kernel(_deg_body, **_DEG_KW)
_prop_sc = pl.kernel(_prop_body, **_PROP_KW)


def _dinv_of(deg_ref):
    dp = deg_ref[...]
    return lax.rsqrt(1.0 + dp[0, :, 0] + dp[1, :, 0])[:, None]


def _t1_body(deg_ref, x_ref, w_ref, g0_ref, g1_ref):
    h = jnp.dot(x_ref[...], w_ref[...], preferred_element_type=_F32)
    g = h * _dinv_of(deg_ref)
    g0_ref[...] = g[:, :FH]
    g1_ref[...] = g[:, FH:]


_t1 = pl.pallas_call(
    _t1_body,
    grid=(GRID,),
    in_specs=[
        pl.BlockSpec((NC, RB, DEGW), lambda i: (0, i, 0)),
        pl.BlockSpec((RB, F), lambda i: (i, 0)),
        pl.BlockSpec((F, F), lambda i: (0, 0)),
    ],
    out_specs=(pl.BlockSpec((RB, FH), lambda i: (i, 0)),
               pl.BlockSpec((RB, FH), lambda i: (i, 0))),
    out_shape=(jax.ShapeDtypeStruct((NP, FH), _F32),
               jax.ShapeDtypeStruct((NP, FH), _F32)),
)


def _t2_body(deg_ref, p0_ref, p1_ref, g0_ref, g1_ref, b_ref, w_ref,
             o0_ref, o1_ref):
    dinv = _dinv_of(deg_ref)
    agg0 = p0_ref[...] + g0_ref[...]
    agg1 = p1_ref[...] + g1_ref[...]
    agg = jnp.concatenate([agg0, agg1], axis=1)
    h = jnp.maximum(agg * dinv + b_ref[...], 0.0)
    g2 = jnp.dot(h, w_ref[...], preferred_element_type=_F32) * dinv
    o0_ref[...] = g2[:, :FH]
    o1_ref[...] = g2[:, FH:]


_t2 = pl.pallas_call(
    _t2_body,
    grid=(GRID,),
    in_specs=[
        pl.BlockSpec((NC, RB, DEGW), lambda i: (0, i, 0)),
        pl.BlockSpec((RB, FH), lambda i: (i, 0)),
        pl.BlockSpec((RB, FH), lambda i: (i, 0)),
        pl.BlockSpec((RB, FH), lambda i: (i, 0)),
        pl.BlockSpec((RB, FH), lambda i: (i, 0)),
        pl.BlockSpec((1, F), lambda i: (0, 0)),
        pl.BlockSpec((F, F), lambda i: (0, 0)),
    ],
    out_specs=(pl.BlockSpec((RB, FH), lambda i: (i, 0)),
               pl.BlockSpec((RB, FH), lambda i: (i, 0))),
    out_shape=(jax.ShapeDtypeStruct((NP, FH), _F32),
               jax.ShapeDtypeStruct((NP, FH), _F32)),
)


def _t3_body(deg_ref, p0_ref, p1_ref, g0_ref, g1_ref, b_ref, batch_ref,
             wlin_ref, blin_ref, o_ref, pool_acc, cnt_acc):
    i = pl.program_id(0)
    dinv = _dinv_of(deg_ref)
    agg = jnp.concatenate([p0_ref[...] + g0_ref[...],
                           p1_ref[...] + g1_ref[...]], axis=1)
    h = jnp.maximum(agg * dinv + b_ref[...], 0.0)
    bb = batch_ref[...]
    gid = lax.broadcasted_iota(jnp.int32, (RB, NG), 1)
    onehot = (bb == gid).astype(_F32)
    psum = lax.dot_general(onehot, h, (((0,), (0,)), ((), ())),
                           preferred_element_type=_F32)
    csum = jnp.sum(onehot, axis=0)

    @pl.when(i == 0)
    def _():
        pool_acc[...] = jnp.zeros_like(pool_acc)
        cnt_acc[...] = jnp.zeros_like(cnt_acc)

    pool_acc[...] += psum
    cnt_acc[...] += jnp.broadcast_to(csum[:, None], (NG, F))

    @pl.when(i == GRID - 1)
    def _():
        pooled = pool_acc[...] / jnp.maximum(cnt_acc[...], 1.0)
        o_ref[...] = jnp.dot(pooled, wlin_ref[...], preferred_element_type=_F32) + blin_ref[...]


_t3 = pl.pallas_call(
    _t3_body,
    grid=(GRID,),
    in_specs=[
        pl.BlockSpec((NC, RB, DEGW), lambda i: (0, i, 0)),
        pl.BlockSpec((RB, FH), lambda i: (i, 0)),
        pl.BlockSpec((RB, FH), lambda i: (i, 0)),
        pl.BlockSpec((RB, FH), lambda i: (i, 0)),
        pl.BlockSpec((RB, FH), lambda i: (i, 0)),
        pl.BlockSpec((1, F), lambda i: (0, 0)),
        pl.BlockSpec((RB, 1), lambda i: (i, 0)),
        pl.BlockSpec((F, 2), lambda i: (0, 0)),
        pl.BlockSpec((1, 2), lambda i: (0, 0)),
    ],
    out_specs=pl.BlockSpec((NG, 2), lambda i: (0, 0)),
    out_shape=jax.ShapeDtypeStruct((NG, 2), _F32),
    scratch_shapes=[
        pltpu.VMEM((NG, F), _F32),
        pltpu.VMEM((NG, F), _F32),
    ],
)


@jax.jit
def _impl(x, edge_index, batch, W1, b1, W2, b2, Wlin, blin):
    dst_d = edge_index[1].reshape(NC, NS, NCH_D, CHUNK)
    pad = jnp.full((E_PAD - E,), PADNODE, jnp.int32)
    src_p = jnp.concatenate([edge_index[0], pad]).reshape(NS, NCH_P, CHUNK_P)
    dst_p = jnp.concatenate([edge_index[1], pad]).reshape(NS, NCH_P, CHUNK_P)
    x_pad = jnp.pad(x, ((0, NP - N), (0, 0)))
    batch_pad = jnp.pad(batch, (0, NP - N), constant_values=NG)
    degp = _deg_sc(dst_d)
    g10, g11 = _t1(degp, x_pad, W1)
    p10, p11 = _prop_sc(g10, g11, src_p, dst_p)
    g20, g21 = _t2(degp, p10, p11, g10, g11, b1.reshape(1, F), W2)
    p20, p21 = _prop_sc(g20, g21, src_p, dst_p)
    return _t3(degp, p20, p21, g20, g21, b2.reshape(1, F),
               batch_pad.reshape(NP, 1), Wlin, blin.reshape(1, 2))


def kernel(x, edge_index, batch, W1, b1, W2, b2, Wlin, blin):
    return _impl(x, edge_index, batch, W1, b1, W2, b2, Wlin, blin)


# speedup vs baseline: 1.4613x; 1.4613x over previous
import functools

import jax
import jax.numpy as jnp
from jax import lax
from jax.experimental import pallas as pl
from jax.experimental.pallas import tpu as pltpu
from jax.experimental.pallas import tpu_sc as plsc

N = 10000
NP = 10240
E = 320000
F = 128
FH = F // 2
NG = 64

NC = 2
NS = 16
CHUNK = 80
EPT_D = E // (NC * NS)
NCH_D = EPT_D // CHUNK
CHUNK_P = 80
NCH_P = 252
EPT_P = NCH_P * CHUNK_P
E_PAD = NS * EPT_P
PADNODE = N
RPT = NP // NS
ZROWS = 128
DEGW = 16

RB = 1024
GRID = NP // RB

_F32 = jnp.float32


def _sc_mesh():
    return plsc.VectorSubcoreMesh(core_axis_name="c", subcore_axis_name="s",
                                  num_cores=NC, num_subcores=NS)


_DEG_KW = dict(
    mesh=_sc_mesh(),
    out_type=jax.ShapeDtypeStruct((NC, NP, DEGW), _F32),
    scratch_types=[
        pltpu.VMEM((NCH_D, CHUNK), jnp.int32),
        pltpu.VMEM((CHUNK, DEGW), _F32),
        pltpu.VMEM((RPT, DEGW), _F32),
        pltpu.VMEM_SHARED((NP, DEGW), _F32),
    ],
    compiler_params=pltpu.CompilerParams(use_tc_tiling_on_sc=False),
)


def _deg_body(dst_hbm, out_hbm, dstv, onesv, zb, deg_sh):
    c = lax.axis_index("c")
    s = lax.axis_index("s")
    pltpu.sync_copy(dst_hbm.at[c, s], dstv)

    ones16 = jnp.ones((16,), _F32)
    z16 = jnp.zeros((16,), _F32)

    def fill_ones(r, carry):
        onesv[r, :] = ones16
        return carry

    lax.fori_loop(0, CHUNK, fill_ones, 0)

    def fill_zero(r, carry):
        zb[r, :] = z16
        return carry

    lax.fori_loop(0, RPT, fill_zero, 0)
    pltpu.sync_copy(zb, deg_sh.at[pl.ds(s * RPT, RPT)])
    plsc.subcore_barrier()

    def body(i, carry):
        pltpu.sync_copy(onesv, deg_sh.at[dstv.at[i]], add=True)
        return carry

    lax.fori_loop(0, NCH_D, body, 0)
    plsc.subcore_barrier()
    pltpu.sync_copy(deg_sh.at[pl.ds(s * RPT, RPT)],
                    out_hbm.at[c, pl.ds(s * RPT, RPT)])


_PROP_KW = dict(
    mesh=_sc_mesh(),
    out_type=(jax.ShapeDtypeStruct((NP, FH), _F32),
              jax.ShapeDtypeStruct((NP, FH), _F32)),
    scratch_types=[
        pltpu.VMEM((NCH_P, CHUNK_P), jnp.int32),
        pltpu.VMEM((NCH_P, CHUNK_P), jnp.int32),
        pltpu.VMEM((CHUNK_P, FH), _F32),
        pltpu.VMEM((CHUNK_P, FH), _F32),
        pltpu.VMEM((CHUNK_P, FH), _F32),
        pltpu.VMEM((CHUNK_P, FH), _F32),
        pltpu.VMEM((ZROWS, FH), _F32),
        pltpu.VMEM_SHARED((NP, FH), _F32),
        pltpu.SemaphoreType.DMA,
        pltpu.SemaphoreType.DMA,
        pltpu.SemaphoreType.DMA,
        pltpu.SemaphoreType.DMA,
        pltpu.SemaphoreType.DMA,
        pltpu.SemaphoreType.DMA,
        pltpu.SemaphoreType.DMA,
        pltpu.SemaphoreType.DMA,
    ],
    compiler_params=pltpu.CompilerParams(use_tc_tiling_on_sc=False),
)


def _prop_body(g0_hbm, g1_hbm, src_hbm, dst_hbm, p0_hbm, p1_hbm,
             srcv, dstv, gb0, gb1, gb2, gb3, zb, p_sh,
             sg0, sg1, sg2, sg3, ss0, ss1, ss2, ss3):
    c = lax.axis_index("c")
    s = lax.axis_index("s")
    pltpu.sync_copy(src_hbm.at[s], srcv)
    pltpu.sync_copy(dst_hbm.at[s], dstv)

    z16 = jnp.zeros((16,), _F32)

    def fill_zero(r, carry):
        for j in range(FH // 16):
            zb[r, pl.ds(j * 16, 16)] = z16
        return carry

    lax.fori_loop(0, ZROWS, fill_zero, 0)
    for k in range(RPT // ZROWS):
        pltpu.sync_copy(zb, p_sh.at[pl.ds(s * RPT + k * ZROWS, ZROWS)])
    plsc.subcore_barrier()

    def run_half(g_hbm):
        bufs = (gb0, gb1, gb2, gb3)
        sgs = (sg0, sg1, sg2, sg3)
        sss = (ss0, ss1, ss2, ss3)

        def wait_g(buf, sem):
            pltpu.make_async_copy(g_hbm.at[srcv.at[0]], buf, sem).wait()

        def wait_s(buf, sem):
            pltpu.make_async_copy(buf, p_sh.at[dstv.at[0]], sem).wait()

        pltpu.async_copy(g_hbm.at[srcv.at[0]], bufs[0], sgs[0])
        ngrp = NCH_P // 4

        def body4(j, carry):
            i = 4 * j
            for b in range(4):
                nb = (b + 1) % 4
                if b < 3:
                    @pl.when(j > 0)
                    def _(nb=nb):
                        wait_s(bufs[nb], sss[nb])
                    pltpu.async_copy(g_hbm.at[srcv.at[i + b + 1]],
                                     bufs[nb], sgs[nb])
                else:
                    wait_s(bufs[0], sss[0])

                    @pl.when(j < ngrp - 1)
                    def _():
                        pltpu.async_copy(g_hbm.at[srcv.at[i + 4]],
                                         bufs[0], sgs[0])
                wait_g(bufs[b], sgs[b])
                pltpu.async_copy(bufs[b], p_sh.at[dstv.at[i + b]], sss[b],
                                 add=True)
            return carry

        lax.fori_loop(0, ngrp, body4, 0)
        for b in (1, 2, 3):
            wait_s(bufs[b], sss[b])

    @pl.when(c == 0)
    def _():
        run_half(g0_hbm)

    @pl.when(c == 1)
    def _():
        run_half(g1_hbm)

    plsc.subcore_barrier()

    @pl.when(c == 0)
    def _():
        pltpu.sync_copy(p_sh.at[pl.ds(s * RPT, RPT)],
                        p0_hbm.at[pl.ds(s * RPT, RPT)])

    @pl.when(c == 1)
    def _():
        pltpu.sync_copy(p_sh.at[pl.ds(s * RPT, RPT)],
                        p1_hbm.at[pl.ds(s * RPT, RPT)])


_deg_sc = pl.kernel(_deg_body, **_DEG_KW)
_prop_sc = pl.kernel(_prop_body, **_PROP_KW)


def _dinv_of(deg_ref):
    dp = deg_ref[...]
    return lax.rsqrt(1.0 + dp[0, :, 0] + dp[1, :, 0])[:, None]


def _t1_body(deg_ref, x_ref, w_ref, g0_ref, g1_ref):
    h = jnp.dot(x_ref[...], w_ref[...], preferred_element_type=_F32)
    g = h * _dinv_of(deg_ref)
    g0_ref[...] = g[:, :FH]
    g1_ref[...] = g[:, FH:]


_t1 = pl.pallas_call(
    _t1_body,
    grid=(GRID,),
    in_specs=[
        pl.BlockSpec((NC, RB, DEGW), lambda i: (0, i, 0)),
        pl.BlockSpec((RB, F), lambda i: (i, 0)),
        pl.BlockSpec((F, F), lambda i: (0, 0)),
    ],
    out_specs=(pl.BlockSpec((RB, FH), lambda i: (i, 0)),
               pl.BlockSpec((RB, FH), lambda i: (i, 0))),
    out_shape=(jax.ShapeDtypeStruct((NP, FH), _F32),
               jax.ShapeDtypeStruct((NP, FH), _F32)),
)


def _t2_body(deg_ref, p0_ref, p1_ref, g0_ref, g1_ref, b_ref, w_ref,
             o0_ref, o1_ref):
    dinv = _dinv_of(deg_ref)
    agg0 = p0_ref[...] + g0_ref[...]
    agg1 = p1_ref[...] + g1_ref[...]
    agg = jnp.concatenate([agg0, agg1], axis=1)
    h = jnp.maximum(agg * dinv + b_ref[...], 0.0)
    g2 = jnp.dot(h, w_ref[...], preferred_element_type=_F32) * dinv
    o0_ref[...] = g2[:, :FH]
    o1_ref[...] = g2[:, FH:]


_t2 = pl.pallas_call(
    _t2_body,
    grid=(GRID,),
    in_specs=[
        pl.BlockSpec((NC, RB, DEGW), lambda i: (0, i, 0)),
        pl.BlockSpec((RB, FH), lambda i: (i, 0)),
        pl.BlockSpec((RB, FH), lambda i: (i, 0)),
        pl.BlockSpec((RB, FH), lambda i: (i, 0)),
        pl.BlockSpec((RB, FH), lambda i: (i, 0)),
        pl.BlockSpec((1, F), lambda i: (0, 0)),
        pl.BlockSpec((F, F), lambda i: (0, 0)),
    ],
    out_specs=(pl.BlockSpec((RB, FH), lambda i: (i, 0)),
               pl.BlockSpec((RB, FH), lambda i: (i, 0))),
    out_shape=(jax.ShapeDtypeStruct((NP, FH), _F32),
               jax.ShapeDtypeStruct((NP, FH), _F32)),
)


def _t3_body(deg_ref, p0_ref, p1_ref, g0_ref, g1_ref, b_ref, batch_ref,
             wlin_ref, blin_ref, o_ref, pool_acc, cnt_acc):
    i = pl.program_id(0)
    dinv = _dinv_of(deg_ref)
    agg = jnp.concatenate([p0_ref[...] + g0_ref[...],
                           p1_ref[...] + g1_ref[...]], axis=1)
    h = jnp.maximum(agg * dinv + b_ref[...], 0.0)
    bb = batch_ref[...]
    gid = lax.broadcasted_iota(jnp.int32, (RB, NG), 1)
    onehot = (bb == gid).astype(_F32)
    psum = lax.dot_general(onehot, h, (((0,), (0,)), ((), ())),
                           preferred_element_type=_F32)
    csum = jnp.sum(onehot, axis=0)

    @pl.when(i == 0)
    def _():
        pool_acc[...] = jnp.zeros_like(pool_acc)
        cnt_acc[...] = jnp.zeros_like(cnt_acc)

    pool_acc[...] += psum
    cnt_acc[...] += jnp.broadcast_to(csum[:, None], (NG, F))

    @pl.when(i == GRID - 1)
    def _():
        pooled = pool_acc[...] / jnp.maximum(cnt_acc[...], 1.0)
        o_ref[...] = jnp.dot(pooled, wlin_ref[...], preferred_element_type=_F32) + blin_ref[...]


_t3 = pl.pallas_call(
    _t3_body,
    grid=(GRID,),
    in_specs=[
        pl.BlockSpec((NC, RB, DEGW), lambda i: (0, i, 0)),
        pl.BlockSpec((RB, FH), lambda i: (i, 0)),
        pl.BlockSpec((RB, FH), lambda i: (i, 0)),
        pl.BlockSpec((RB, FH), lambda i: (i, 0)),
        pl.BlockSpec((RB, FH), lambda i: (i, 0)),
        pl.BlockSpec((1, F), lambda i: (0, 0)),
        pl.BlockSpec((RB, 1), lambda i: (i, 0)),
        pl.BlockSpec((F, 2), lambda i: (0, 0)),
        pl.BlockSpec((1, 2), lambda i: (0, 0)),
    ],
    out_specs=pl.BlockSpec((NG, 2), lambda i: (0, 0)),
    out_shape=jax.ShapeDtypeStruct((NG, 2), _F32),
    scratch_shapes=[
        pltpu.VMEM((NG, F), _F32),
        pltpu.VMEM((NG, F), _F32),
    ],
)


@jax.jit
def _impl(x, edge_index, batch, W1, b1, W2, b2, Wlin, blin):
    dst_d = edge_index[1].reshape(NC, NS, NCH_D, CHUNK)
    pad = jnp.full((E_PAD - E,), PADNODE, jnp.int32)
    src_p = jnp.concatenate([edge_index[0], pad]).reshape(NS, NCH_P, CHUNK_P)
    dst_p = jnp.concatenate([edge_index[1], pad]).reshape(NS, NCH_P, CHUNK_P)
    x_pad = jnp.pad(x, ((0, NP - N), (0, 0)))
    batch_pad = jnp.pad(batch, (0, NP - N), constant_values=NG)
    degp = _deg_sc(dst_d)
    g10, g11 = _t1(degp, x_pad, W1)
    p10, p11 = _prop_sc(g10, g11, src_p, dst_p)
    g20, g21 = _t2(degp, p10, p11, g10, g11, b1.reshape(1, F), W2)
    p20, p21 = _prop_sc(g20, g21, src_p, dst_p)
    return _t3(degp, p20, p21, g20, g21, b2.reshape(1, F),
               batch_pad.reshape(NP, 1), Wlin, blin.reshape(1, 2))


def kernel(x, edge_index, batch, W1, b1, W2, b2, Wlin, blin):
    return _impl(x, edge_index, batch, W1, b1, W2, b2, Wlin, blin)
